# Initial kernel scaffold; baseline (speedup 1.0000x reference)
import jax
import jax.numpy as jnp
from jax.experimental import pallas as pl


def kernel(x, W):
    raise NotImplementedError("write your pallas kernel here")


# speedup vs baseline: 3.9586x; 3.9586x over previous
import functools

import jax
import jax.numpy as jnp
from jax import lax
from jax.experimental import pallas as pl
from jax.experimental.pallas import tpu as pltpu
from jax.experimental.pallas import tpu_sc as plsc

C_IN = 100000
D_MODEL = 64
BATCH = 4096
SEQ = 200
B_TOTAL = BATCH * SEQ

_info = plsc.get_sparse_core_info()
NC = _info.num_cores
NS = _info.num_subcores
NW = NC * NS
B_PER_W = B_TOTAL // NW

CHUNK = 512
N_CHUNKS = B_PER_W // CHUNK


def _gather_kernel(x_hbm, w_hbm, out_hbm, idx_v, rows_v, sem):
    wid = lax.axis_index("s") * NC + lax.axis_index("c")
    base = wid * B_PER_W

    def chunk_body(g, carry):
        off = base + g * CHUNK
        pltpu.sync_copy(x_hbm.at[pl.ds(off, CHUNK)], idx_v)
        pltpu.async_copy(w_hbm.at[idx_v], rows_v, sem).wait()
        pltpu.sync_copy(rows_v, out_hbm.at[pl.ds(off, CHUNK)])
        return carry

    lax.fori_loop(0, N_CHUNKS, chunk_body, 0)


@jax.jit
def _embed(x_flat, W):
    mesh = plsc.VectorSubcoreMesh(core_axis_name="c", subcore_axis_name="s")
    run = functools.partial(
        pl.kernel,
        mesh=mesh,
        out_type=jax.ShapeDtypeStruct((B_TOTAL, D_MODEL), jnp.float32),
        scratch_types=[
            pltpu.VMEM((CHUNK,), jnp.int32),
            pltpu.VMEM((CHUNK, D_MODEL), jnp.float32),
            pltpu.SemaphoreType.DMA,
        ],
        compiler_params=pltpu.CompilerParams(use_tc_tiling_on_sc=False),
    )(_gather_kernel)
    return run(x_flat, W)


def kernel(x, W):
    out = _embed(x.reshape(-1), W)
    return out.reshape(BATCH, SEQ, D_MODEL)


# speedup vs baseline: 4.2480x; 1.0731x over previous
import functools

import jax
import jax.numpy as jnp
from jax import lax
from jax.experimental import pallas as pl
from jax.experimental.pallas import tpu as pltpu
from jax.experimental.pallas import tpu_sc as plsc

C_IN = 100000
D_MODEL = 64
BATCH = 4096
SEQ = 200
B_TOTAL = BATCH * SEQ

_info = plsc.get_sparse_core_info()
NC = _info.num_cores
NS = _info.num_subcores
NW = NC * NS
B_PER_W = B_TOTAL // NW

CHUNK = 800
N_CHUNKS = B_PER_W // CHUNK
NBUF = 2


def _gather_kernel(x_hbm, w_hbm, out_hbm, idx_v, rows_v,
                   sem_idx, sem_g, sem_w):
    wid = lax.axis_index("s") * NC + lax.axis_index("c")
    base = wid * B_PER_W

    for b in range(NBUF):
        pltpu.async_copy(
            x_hbm.at[pl.ds(base + b * CHUNK, CHUNK)], idx_v.at[b],
            sem_idx.at[b])

    def super_body(s, carry):
        for b in range(NBUF):
            g = s * NBUF + b
            off = base + g * CHUNK
            @pl.when(s > 0)
            def _():
                pltpu.make_async_copy(
                    rows_v.at[b], out_hbm.at[pl.ds(off, CHUNK)],
                    sem_w.at[b]).wait()
            pltpu.make_async_copy(
                x_hbm.at[pl.ds(off, CHUNK)], idx_v.at[b],
                sem_idx.at[b]).wait()
            pltpu.async_copy(w_hbm.at[idx_v.at[b]], rows_v.at[b],
                             sem_g.at[b]).wait()
            @pl.when(g + NBUF < N_CHUNKS)
            def _():
                pltpu.async_copy(
                    x_hbm.at[pl.ds(off + NBUF * CHUNK, CHUNK)],
                    idx_v.at[b], sem_idx.at[b])
            pltpu.async_copy(rows_v.at[b], out_hbm.at[pl.ds(off, CHUNK)],
                             sem_w.at[b])
        return carry

    lax.fori_loop(0, N_CHUNKS // NBUF, super_body, 0)

    for b in range(NBUF):
        off = base + (N_CHUNKS - NBUF + b) * CHUNK
        pltpu.make_async_copy(
            rows_v.at[b], out_hbm.at[pl.ds(off, CHUNK)], sem_w.at[b]).wait()


@jax.jit
def _embed(x_flat, W):
    mesh = plsc.VectorSubcoreMesh(core_axis_name="c", subcore_axis_name="s")
    run = functools.partial(
        pl.kernel,
        mesh=mesh,
        out_type=jax.ShapeDtypeStruct((B_TOTAL, D_MODEL), jnp.float32),
        scratch_types=[
            pltpu.VMEM((NBUF, CHUNK), jnp.int32),
            pltpu.VMEM((NBUF, CHUNK, D_MODEL), jnp.float32),
            pltpu.SemaphoreType.DMA((NBUF,)),
            pltpu.SemaphoreType.DMA((NBUF,)),
            pltpu.SemaphoreType.DMA((NBUF,)),
        ],
        compiler_params=pltpu.CompilerParams(use_tc_tiling_on_sc=False),
    )(_gather_kernel)
    return run(x_flat, W)


def kernel(x, W):
    out = _embed(x.reshape(-1), W)
    return out.reshape(BATCH, SEQ, D_MODEL)
